# Initial kernel scaffold; baseline (speedup 1.0000x reference)
import jax
import jax.numpy as jnp
from jax.experimental import pallas as pl


def kernel(h, edge_index, emb_table, alphas_first, alphas_middle, alphas_last, W_dense, b_dense, W_fc0, b_fc0, W_fc1, b_fc1, W_fc2, b_fc2):
    raise NotImplementedError("write your pallas kernel here")


# speedup vs baseline: 7.4538x; 7.4538x over previous
import functools

import jax
import jax.numpy as jnp
from jax import lax
from jax.experimental import pallas as pl
from jax.experimental.pallas import tpu as pltpu
from jax.experimental.pallas import tpu_sc as plsc

_N = 10000
_E = 320000
_D = 128
_C = 40
_NSC = 2
_NT = 16
_NW = _NSC * _NT
_EPR = 2560
_RPW = _EPR // _NW
_APAD = 10240
_APT = _APAD // _NT


@functools.lru_cache(maxsize=None)
def _mesh():
    return plsc.VectorSubcoreMesh(
        core_axis_name="c", subcore_axis_name="s",
        num_cores=_NSC, num_subcores=_NT)


def _emb_gather(h1, emb):
    @functools.partial(
        pl.kernel,
        out_type=jax.ShapeDtypeStruct((_APAD, _D), jnp.float32),
        mesh=_mesh(),
        scratch_types=[
            pltpu.VMEM((128,), jnp.int32),
            pltpu.VMEM((128, _D), jnp.float32),
            pltpu.SemaphoreType.DMA,
        ],
    )
    def k(h1_hbm, emb_hbm, out_hbm, idxv, rows, sem):
        cid = lax.axis_index("c")
        sid = lax.axis_index("s")
        wid = sid * _NSC + cid

        def do(kk):
            pltpu.sync_copy(h1_hbm.at[pl.ds(kk * 128, 128)], idxv)
            pltpu.async_copy(emb_hbm.at[idxv], rows, sem).wait()
            pltpu.sync_copy(rows, out_hbm.at[pl.ds(kk * 128, 128)])

        do(wid)
        do(wid + _NW)

        @pl.when(wid < (_APAD // 128) - 2 * _NW)
        def _():
            do(wid + 2 * _NW)

    return k(h1, emb)


def _deg_hist(dst2):
    @functools.partial(
        pl.kernel,
        out_type=jax.ShapeDtypeStruct((_NSC, _APAD), jnp.float32),
        mesh=_mesh(),
        scratch_types=[
            pltpu.VMEM((_RPW, 128), jnp.int32),
            pltpu.VMEM((128,), jnp.float32),
            pltpu.VMEM((_APT,), jnp.float32),
            pltpu.VMEM_SHARED((_APAD,), jnp.float32),
        ],
    )
    def k(dst_hbm, out_hbm, dstv, onesv, zb, dacc):
        cid = lax.axis_index("c")
        sid = lax.axis_index("s")
        wid = sid * _NSC + cid
        ones16 = jnp.full((16,), 1.0, jnp.float32)
        z16 = jnp.zeros((16,), jnp.float32)

        def f1(i, _):
            onesv[pl.ds(i * 16, 16)] = ones16
            return 0

        lax.fori_loop(0, 128 // 16, f1, 0)

        def f0(i, _):
            zb[pl.ds(i * 16, 16)] = z16
            return 0

        lax.fori_loop(0, _APT // 16, f0, 0)
        pltpu.sync_copy(zb, dacc.at[pl.ds(sid * _APT, _APT)])
        plsc.subcore_barrier()

        pltpu.sync_copy(dst_hbm.at[pl.ds(wid * _RPW, _RPW)], dstv)

        def body(j, _):
            pltpu.sync_copy(onesv, dacc.at[dstv.at[j]], add=True)
            return 0

        lax.fori_loop(0, _RPW, body, 0)

        plsc.subcore_barrier()
        pltpu.sync_copy(dacc.at[pl.ds(sid * _APT, _APT)],
                        out_hbm.at[cid, pl.ds(sid * _APT, _APT)])

    return k(dst2)


def _segsum(z, src2, dst2):
    @functools.partial(
        pl.kernel,
        out_type=jax.ShapeDtypeStruct((_NSC, _N, _D), jnp.float32),
        mesh=_mesh(),
        scratch_types=[
            pltpu.VMEM((_RPW, 128), jnp.int32),
            pltpu.VMEM((_RPW, 128), jnp.int32),
            pltpu.VMEM((128, _D), jnp.float32),
            pltpu.SemaphoreType.DMA,
            pltpu.VMEM_SHARED((_APAD, _D), jnp.float32),
        ],
    )
    def k(z_hbm, src_hbm, dst_hbm, out_hbm, srcv, dstv, rows, sem, acc):
        cid = lax.axis_index("c")
        sid = lax.axis_index("s")
        wid = sid * _NSC + cid
        z16 = jnp.zeros((16,), jnp.float32)

        def zf(i, _):
            rows[i // (_D // 16), pl.ds((i % (_D // 16)) * 16, 16)] = z16
            return 0

        lax.fori_loop(0, 128 * (_D // 16), zf, 0)
        for t in range(_APT // 128):
            pltpu.sync_copy(rows, acc.at[pl.ds(sid * _APT + t * 128, 128)])
        plsc.subcore_barrier()

        pltpu.sync_copy(src_hbm.at[pl.ds(wid * _RPW, _RPW)], srcv)
        pltpu.sync_copy(dst_hbm.at[pl.ds(wid * _RPW, _RPW)], dstv)

        def body(j, _):
            pltpu.async_copy(z_hbm.at[srcv.at[j]], rows, sem).wait()
            pltpu.sync_copy(rows, acc.at[dstv.at[j]], add=True)
            return 0

        lax.fori_loop(0, _RPW, body, 0)

        plsc.subcore_barrier()
        @pl.when(sid < _NT - 1)
        def _():
            pltpu.sync_copy(acc.at[pl.ds(sid * _APT, _APT)],
                            out_hbm.at[cid, pl.ds(sid * _APT, _APT)])

        @pl.when(sid == _NT - 1)
        def _():
            pltpu.sync_copy(acc.at[pl.ds((_NT - 1) * _APT, _N - (_NT - 1) * _APT)],
                            out_hbm.at[cid, pl.ds((_NT - 1) * _APT,
                                                  _N - (_NT - 1) * _APT)])

    return k(z, src2, dst2)


_BN = 1000
_GRID = _N // _BN


def _row_spec():
    return pl.BlockSpec((_BN, _D), lambda i: (i, 0))


_W_spec = pl.BlockSpec((_D, _D), lambda i: (0, 0))
_b_spec = pl.BlockSpec((1, _D), lambda i: (0, 0))
_dinv_spec = pl.BlockSpec((_BN, 1), lambda i: (i, 0))
_scal_spec = pl.BlockSpec(memory_space=pltpu.MemorySpace.SMEM)


def _mm_relu(z, W, b):
    return jnp.maximum(jnp.dot(z, W, preferred_element_type=jnp.float32) + b, 0.0)


def _stageA(x, p0, p1, dinv, W, b, scal):
    def body(scal_ref, x_ref, p0_ref, p1_ref, dinv_ref, W_ref, b_ref,
             s1_ref, s2p_ref):
        x = x_ref[...]
        dx = _mm_relu(x, W_ref[...], b_ref[...])
        sxn = (p0_ref[...] + p1_ref[...]) * dinv_ref[...]
        s1_ref[...] = scal_ref[0] * x + scal_ref[1] * dx + scal_ref[2] * sxn
        s2p_ref[...] = scal_ref[3] * x + scal_ref[4] * dx + scal_ref[5] * sxn

    return pl.pallas_call(
        body,
        grid=(_GRID,),
        in_specs=[_scal_spec, _row_spec(), _row_spec(), _row_spec(),
                  _dinv_spec, _W_spec, _b_spec],
        out_specs=[_row_spec(), _row_spec()],
        out_shape=[jax.ShapeDtypeStruct((_N, _D), jnp.float32)] * 2,
    )(scal, x, p0, p1, dinv, W, b)


def _stageB(s1, p0, p1, dinv, s2p, W, b, scal):
    def body(scal_ref, s1_ref, p0_ref, p1_ref, dinv_ref, s2p_ref, W_ref, b_ref,
             s2_ref, m0_ref):
        s1 = s1_ref[...]
        ds1 = _mm_relu(s1, W_ref[...], b_ref[...])
        ss1n = (p0_ref[...] + p1_ref[...]) * dinv_ref[...]
        s2_ref[...] = (s2p_ref[...] + scal_ref[0] * s1 + scal_ref[1] * ds1
                       + scal_ref[2] * ss1n)
        m0_ref[...] = scal_ref[3] * s1 + scal_ref[4] * ss1n

    return pl.pallas_call(
        body,
        grid=(_GRID,),
        in_specs=[_scal_spec, _row_spec(), _row_spec(), _row_spec(),
                  _dinv_spec, _row_spec(), _W_spec, _b_spec],
        out_specs=[_row_spec(), _row_spec()],
        out_shape=[jax.ShapeDtypeStruct((_N, _D), jnp.float32)] * 2,
    )(scal, s1, p0, p1, dinv, s2p, W, b)


def _stageC(s2, q0, q1, r0, r1, m0, dinv, W, b, scal):
    def body(scal_ref, s2_ref, q0_ref, q1_ref, r0_ref, r1_ref, m0_ref,
             dinv_ref, W_ref, b_ref, m1_ref, l0p_ref, l1p_ref):
        s2 = s2_ref[...]
        m0 = m0_ref[...]
        dinv = dinv_ref[...]
        ss2n = (q0_ref[...] + q1_ref[...]) * dinv
        sm0n = (r0_ref[...] + r1_ref[...]) * dinv
        dm0 = _mm_relu(m0, W_ref[...], b_ref[...])
        m1_ref[...] = scal_ref[0] * s2 + scal_ref[1] * ss2n
        l0p_ref[...] = scal_ref[2] * m0 + scal_ref[3] * dm0 + scal_ref[4] * sm0n
        l1p_ref[...] = scal_ref[5] * m0 + scal_ref[6] * dm0 + scal_ref[7] * sm0n

    return pl.pallas_call(
        body,
        grid=(_GRID,),
        in_specs=[_scal_spec, _row_spec(), _row_spec(), _row_spec(),
                  _row_spec(), _row_spec(), _row_spec(), _dinv_spec,
                  _W_spec, _b_spec],
        out_specs=[_row_spec(), _row_spec(), _row_spec()],
        out_shape=[jax.ShapeDtypeStruct((_N, _D), jnp.float32)] * 3,
    )(scal, s2, q0, q1, r0, r1, m0, dinv, W, b)


def _stageD(m1, t0, t1, l0p, l1p, dinv, W, b, scal):
    def body(scal_ref, m1_ref, t0_ref, t1_ref, l0p_ref, l1p_ref, dinv_ref,
             W_ref, b_ref, l0_ref, l1q_ref):
        m1 = m1_ref[...]
        sm1n = (t0_ref[...] + t1_ref[...]) * dinv_ref[...]
        dm1 = _mm_relu(m1, W_ref[...], b_ref[...])
        l0_ref[...] = (l0p_ref[...] + scal_ref[0] * m1 + scal_ref[1] * dm1
                       + scal_ref[2] * sm1n)
        l1q_ref[...] = (l1p_ref[...] + scal_ref[3] * m1 + scal_ref[4] * dm1
                        + scal_ref[5] * sm1n)

    return pl.pallas_call(
        body,
        grid=(_GRID,),
        in_specs=[_scal_spec, _row_spec(), _row_spec(), _row_spec(),
                  _row_spec(), _row_spec(), _dinv_spec, _W_spec, _b_spec],
        out_specs=[_row_spec(), _row_spec()],
        out_shape=[jax.ShapeDtypeStruct((_N, _D), jnp.float32)] * 2,
    )(scal, m1, t0, t1, l0p, l1p, dinv, W, b)


def _stageE(l0, l1q, u0, u1, dinv, W, b, W0, b0, W1, b1, W2, b2, scal):
    def body(scal_ref, l0_ref, l1q_ref, u0_ref, u1_ref, dinv_ref, W_ref, b_ref,
             W0_ref, b0_ref, W1_ref, b1_ref, W2_ref, b2_ref, y_ref):
        l0 = l0_ref[...]
        sl0n = (u0_ref[...] + u1_ref[...]) * dinv_ref[...]
        dl0 = _mm_relu(l0, W_ref[...], b_ref[...])
        l1 = (l1q_ref[...] + scal_ref[0] * l0 + scal_ref[1] * dl0
              + scal_ref[2] * sl0n)
        hout = 0.5 * (l0 + l1)
        y0 = _mm_relu(hout, W0_ref[...], b0_ref[...])
        y1 = _mm_relu(y0, W1_ref[...], b1_ref[...])
        y_ref[...] = jnp.dot(y1, W2_ref[...],
                             preferred_element_type=jnp.float32) + b2_ref[...]

    return pl.pallas_call(
        body,
        grid=(_GRID,),
        in_specs=[
            _scal_spec, _row_spec(), _row_spec(), _row_spec(), _row_spec(),
            _dinv_spec, _W_spec, _b_spec,
            pl.BlockSpec((_D, _D // 2), lambda i: (0, 0)),
            pl.BlockSpec((1, _D // 2), lambda i: (0, 0)),
            pl.BlockSpec((_D // 2, _D // 4), lambda i: (0, 0)),
            pl.BlockSpec((1, _D // 4), lambda i: (0, 0)),
            pl.BlockSpec((_D // 4, _C), lambda i: (0, 0)),
            pl.BlockSpec((1, _C), lambda i: (0, 0)),
        ],
        out_specs=pl.BlockSpec((_BN, _C), lambda i: (i, 0)),
        out_shape=jax.ShapeDtypeStruct((_N, _C), jnp.float32),
    )(scal, l0, l1q, u0, u1, dinv, W, b, W0, b0, W1, b1, W2, b2)


def kernel(h, edge_index, emb_table, alphas_first, alphas_middle, alphas_last,
           W_dense, b_dense, W_fc0, b_fc0, W_fc1, b_fc1, W_fc2, b_fc2):
    h = h.astype(jnp.int32)
    pad = _EPR * 128 - _E
    ar = jnp.arange(pad, dtype=jnp.int32)
    src1 = jnp.concatenate([edge_index[0].astype(jnp.int32), ar % _N])
    dst1 = jnp.concatenate([edge_index[1].astype(jnp.int32),
                            _N + (ar % (_APAD - _N))])
    src2 = src1.reshape(_EPR, 128)
    dst2 = dst1.reshape(_EPR, 128)
    ah = jnp.arange(_APAD - _N, dtype=jnp.int32)
    h1 = jnp.concatenate([h, ah % _N])

    Wf = jax.nn.softmax(alphas_first, axis=1)
    Wm = jax.nn.softmax(alphas_middle, axis=1)
    Wl = jax.nn.softmax(alphas_last, axis=1)

    xp = _emb_gather(h1, emb_table)
    x = xp[:_N]
    degp = _deg_hist(dst2)
    dinv = (1.0 / jnp.maximum(degp[0, :_N] + degp[1, :_N], 1.0))[:, None]
    b2d = b_dense[None, :]

    sx = _segsum(x, src2, dst2)
    scalA = jnp.stack([Wf[0, 1], Wf[0, 2], Wf[0, 3],
                       Wf[1, 1], Wf[1, 2], Wf[1, 3]])
    s1, s2p = _stageA(x, sx[0], sx[1], dinv, W_dense, b2d, scalA)

    ss1 = _segsum(s1, src2, dst2)
    scalB = jnp.stack([Wf[2, 1], Wf[2, 2], Wf[2, 3], Wm[0, 0], Wm[0, 1]])
    s2, m0 = _stageB(s1, ss1[0], ss1[1], dinv, s2p, W_dense, b2d, scalB)

    ss2 = _segsum(s2, src2, dst2)
    sm0 = _segsum(m0, src2, dst2)
    scalC = jnp.stack([Wm[1, 0], Wm[1, 1], Wl[0, 1], Wl[0, 2], Wl[0, 3],
                       Wl[2, 1], Wl[2, 2], Wl[2, 3]])
    m1, l0p, l1p = _stageC(s2, ss2[0], ss2[1], sm0[0], sm0[1], m0, dinv,
                           W_dense, b2d, scalC)

    sm1 = _segsum(m1, src2, dst2)
    scalD = jnp.stack([Wl[1, 1], Wl[1, 2], Wl[1, 3],
                       Wl[3, 1], Wl[3, 2], Wl[3, 3]])
    l0, l1q = _stageD(m1, sm1[0], sm1[1], l0p, l1p, dinv, W_dense, b2d, scalD)

    sl0 = _segsum(l0, src2, dst2)
    scalE = jnp.stack([Wl[4, 1], Wl[4, 2], Wl[4, 3]])
    y = _stageE(l0, l1q, sl0[0], sl0[1], dinv, W_dense, b2d,
                W_fc0, b_fc0[None, :], W_fc1, b_fc1[None, :],
                W_fc2, b_fc2[None, :], scalE)
    return y


# speedup vs baseline: 10.9607x; 1.4705x over previous
import functools

import jax
import jax.numpy as jnp
from jax import lax
from jax.experimental import pallas as pl
from jax.experimental.pallas import tpu as pltpu
from jax.experimental.pallas import tpu_sc as plsc

_N = 10000
_E = 320000
_D = 128
_C = 40
_NSC = 2
_NT = 16
_NW = _NSC * _NT
_EPR = 2560
_RPW = _EPR // _NW
_APAD = 10240
_APT = _APAD // _NT


@functools.lru_cache(maxsize=None)
def _mesh():
    return plsc.VectorSubcoreMesh(
        core_axis_name="c", subcore_axis_name="s",
        num_cores=_NSC, num_subcores=_NT)


def _emb_gather(h1, emb):
    @functools.partial(
        pl.kernel,
        out_type=jax.ShapeDtypeStruct((_APAD, _D), jnp.float32),
        mesh=_mesh(),
        scratch_types=[
            pltpu.VMEM((128,), jnp.int32),
            pltpu.VMEM((128, _D), jnp.float32),
            pltpu.SemaphoreType.DMA,
        ],
    )
    def k(h1_hbm, emb_hbm, out_hbm, idxv, rows, sem):
        cid = lax.axis_index("c")
        sid = lax.axis_index("s")
        wid = sid * _NSC + cid

        def do(kk):
            pltpu.sync_copy(h1_hbm.at[pl.ds(kk * 128, 128)], idxv)
            pltpu.async_copy(emb_hbm.at[idxv], rows, sem).wait()
            pltpu.sync_copy(rows, out_hbm.at[pl.ds(kk * 128, 128)])

        do(wid)
        do(wid + _NW)

        @pl.when(wid < (_APAD // 128) - 2 * _NW)
        def _():
            do(wid + 2 * _NW)

    return k(h1, emb)


def _deg_hist(dst2):
    @functools.partial(
        pl.kernel,
        out_type=jax.ShapeDtypeStruct((_NSC, _APAD), jnp.float32),
        mesh=_mesh(),
        scratch_types=[
            pltpu.VMEM((_RPW, 128), jnp.int32),
            pltpu.VMEM((128,), jnp.float32),
            pltpu.VMEM((_APT,), jnp.float32),
            pltpu.VMEM_SHARED((_APAD,), jnp.float32),
        ],
    )
    def k(dst_hbm, out_hbm, dstv, onesv, zb, dacc):
        cid = lax.axis_index("c")
        sid = lax.axis_index("s")
        wid = sid * _NSC + cid
        ones16 = jnp.full((16,), 1.0, jnp.float32)
        z16 = jnp.zeros((16,), jnp.float32)

        def f1(i, _):
            onesv[pl.ds(i * 16, 16)] = ones16
            return 0

        lax.fori_loop(0, 128 // 16, f1, 0)

        def f0(i, _):
            zb[pl.ds(i * 16, 16)] = z16
            return 0

        lax.fori_loop(0, _APT // 16, f0, 0)
        pltpu.sync_copy(zb, dacc.at[pl.ds(sid * _APT, _APT)])
        plsc.subcore_barrier()

        pltpu.sync_copy(dst_hbm.at[pl.ds(wid * _RPW, _RPW)], dstv)

        def body(j, _):
            pltpu.sync_copy(onesv, dacc.at[dstv.at[j]], add=True)
            return 0

        lax.fori_loop(0, _RPW, body, 0)

        plsc.subcore_barrier()
        pltpu.sync_copy(dacc.at[pl.ds(sid * _APT, _APT)],
                        out_hbm.at[cid, pl.ds(sid * _APT, _APT)])

    return k(dst2)


def _segsum(z, src2, dst2):
    @functools.partial(
        pl.kernel,
        out_type=jax.ShapeDtypeStruct((_NSC, _N, _D), jnp.float32),
        mesh=_mesh(),
        scratch_types=[
            pltpu.VMEM((_RPW // 2, 128), jnp.int32),
            pltpu.VMEM((_RPW // 2, 128), jnp.int32),
            pltpu.VMEM((128, _D), jnp.float32),
            pltpu.VMEM((128, _D), jnp.float32),
            pltpu.SemaphoreType.DMA,
            pltpu.SemaphoreType.DMA,
            pltpu.VMEM_SHARED((_APAD, _D), jnp.float32),
        ],
    )
    def k(z_hbm, src_hbm, dst_hbm, out_hbm, srcv, dstv, rows0, rows1,
          sem0, sem1, acc):
        cid = lax.axis_index("c")
        sid = lax.axis_index("s")
        wid = sid * _NSC + cid
        z16 = jnp.zeros((16,), jnp.float32)

        def zf(i, _):
            rows0[i // (_D // 16), pl.ds((i % (_D // 16)) * 16, 16)] = z16
            return 0

        lax.fori_loop(0, 128 * (_D // 16), zf, 0)
        for t in range(_APT // 128):
            pltpu.sync_copy(rows0, acc.at[pl.ds(sid * _APT + t * 128, 128)])
        plsc.subcore_barrier()

        half_rows = _RPW // 2
        for half in range(2):
            rbase = wid * _RPW + half * half_rows
            pltpu.sync_copy(src_hbm.at[pl.ds(rbase, half_rows)], srcv)
            pltpu.sync_copy(dst_hbm.at[pl.ds(rbase, half_rows)], dstv)
            pltpu.async_copy(z_hbm.at[srcv.at[0]], rows0, sem0)

            def body(g, _):
                j0 = g * 2
                pltpu.async_copy(z_hbm.at[srcv.at[j0 + 1]], rows1, sem1)
                pltpu.make_async_copy(z_hbm.at[srcv.at[j0]], rows0, sem0).wait()
                pltpu.sync_copy(rows0, acc.at[dstv.at[j0]], add=True)

                @pl.when(g < half_rows // 2 - 1)
                def _():
                    pltpu.async_copy(z_hbm.at[srcv.at[j0 + 2]], rows0, sem0)

                pltpu.make_async_copy(z_hbm.at[srcv.at[j0 + 1]], rows1,
                                      sem1).wait()
                pltpu.sync_copy(rows1, acc.at[dstv.at[j0 + 1]], add=True)
                return 0

            lax.fori_loop(0, half_rows // 2, body, 0)

        plsc.subcore_barrier()
        @pl.when(sid < _NT - 1)
        def _():
            pltpu.sync_copy(acc.at[pl.ds(sid * _APT, _APT)],
                            out_hbm.at[cid, pl.ds(sid * _APT, _APT)])

        @pl.when(sid == _NT - 1)
        def _():
            pltpu.sync_copy(acc.at[pl.ds((_NT - 1) * _APT, _N - (_NT - 1) * _APT)],
                            out_hbm.at[cid, pl.ds((_NT - 1) * _APT,
                                                  _N - (_NT - 1) * _APT)])

    return k(z, src2, dst2)


_BN = 1000
_GRID = _N // _BN


def _row_spec():
    return pl.BlockSpec((_BN, _D), lambda i: (i, 0))


_W_spec = pl.BlockSpec((_D, _D), lambda i: (0, 0))
_b_spec = pl.BlockSpec((1, _D), lambda i: (0, 0))
_dinv_spec = pl.BlockSpec((_BN, 1), lambda i: (i, 0))
_scal_spec = pl.BlockSpec(memory_space=pltpu.MemorySpace.SMEM)


def _mm_relu(z, W, b):
    return jnp.maximum(jnp.dot(z, W, preferred_element_type=jnp.float32) + b, 0.0)


def _stageA(x, p0, p1, dinv, W, b, scal):
    def body(scal_ref, x_ref, p0_ref, p1_ref, dinv_ref, W_ref, b_ref,
             s1_ref, s2p_ref):
        x = x_ref[...]
        dx = _mm_relu(x, W_ref[...], b_ref[...])
        sxn = (p0_ref[...] + p1_ref[...]) * dinv_ref[...]
        s1_ref[...] = scal_ref[0] * x + scal_ref[1] * dx + scal_ref[2] * sxn
        s2p_ref[...] = scal_ref[3] * x + scal_ref[4] * dx + scal_ref[5] * sxn

    return pl.pallas_call(
        body,
        grid=(_GRID,),
        in_specs=[_scal_spec, _row_spec(), _row_spec(), _row_spec(),
                  _dinv_spec, _W_spec, _b_spec],
        out_specs=[_row_spec(), _row_spec()],
        out_shape=[jax.ShapeDtypeStruct((_N, _D), jnp.float32)] * 2,
    )(scal, x, p0, p1, dinv, W, b)


def _stageB(s1, p0, p1, dinv, s2p, W, b, scal):
    def body(scal_ref, s1_ref, p0_ref, p1_ref, dinv_ref, s2p_ref, W_ref, b_ref,
             s2_ref, m0_ref):
        s1 = s1_ref[...]
        ds1 = _mm_relu(s1, W_ref[...], b_ref[...])
        ss1n = (p0_ref[...] + p1_ref[...]) * dinv_ref[...]
        s2_ref[...] = (s2p_ref[...] + scal_ref[0] * s1 + scal_ref[1] * ds1
                       + scal_ref[2] * ss1n)
        m0_ref[...] = scal_ref[3] * s1 + scal_ref[4] * ss1n

    return pl.pallas_call(
        body,
        grid=(_GRID,),
        in_specs=[_scal_spec, _row_spec(), _row_spec(), _row_spec(),
                  _dinv_spec, _row_spec(), _W_spec, _b_spec],
        out_specs=[_row_spec(), _row_spec()],
        out_shape=[jax.ShapeDtypeStruct((_N, _D), jnp.float32)] * 2,
    )(scal, s1, p0, p1, dinv, s2p, W, b)


def _stageC(s2, q0, q1, r0, r1, m0, dinv, W, b, scal):
    def body(scal_ref, s2_ref, q0_ref, q1_ref, r0_ref, r1_ref, m0_ref,
             dinv_ref, W_ref, b_ref, m1_ref, l0p_ref, l1p_ref):
        s2 = s2_ref[...]
        m0 = m0_ref[...]
        dinv = dinv_ref[...]
        ss2n = (q0_ref[...] + q1_ref[...]) * dinv
        sm0n = (r0_ref[...] + r1_ref[...]) * dinv
        dm0 = _mm_relu(m0, W_ref[...], b_ref[...])
        m1_ref[...] = scal_ref[0] * s2 + scal_ref[1] * ss2n
        l0p_ref[...] = scal_ref[2] * m0 + scal_ref[3] * dm0 + scal_ref[4] * sm0n
        l1p_ref[...] = scal_ref[5] * m0 + scal_ref[6] * dm0 + scal_ref[7] * sm0n

    return pl.pallas_call(
        body,
        grid=(_GRID,),
        in_specs=[_scal_spec, _row_spec(), _row_spec(), _row_spec(),
                  _row_spec(), _row_spec(), _row_spec(), _dinv_spec,
                  _W_spec, _b_spec],
        out_specs=[_row_spec(), _row_spec(), _row_spec()],
        out_shape=[jax.ShapeDtypeStruct((_N, _D), jnp.float32)] * 3,
    )(scal, s2, q0, q1, r0, r1, m0, dinv, W, b)


def _stageD(m1, t0, t1, l0p, l1p, dinv, W, b, scal):
    def body(scal_ref, m1_ref, t0_ref, t1_ref, l0p_ref, l1p_ref, dinv_ref,
             W_ref, b_ref, l0_ref, l1q_ref):
        m1 = m1_ref[...]
        sm1n = (t0_ref[...] + t1_ref[...]) * dinv_ref[...]
        dm1 = _mm_relu(m1, W_ref[...], b_ref[...])
        l0_ref[...] = (l0p_ref[...] + scal_ref[0] * m1 + scal_ref[1] * dm1
                       + scal_ref[2] * sm1n)
        l1q_ref[...] = (l1p_ref[...] + scal_ref[3] * m1 + scal_ref[4] * dm1
                        + scal_ref[5] * sm1n)

    return pl.pallas_call(
        body,
        grid=(_GRID,),
        in_specs=[_scal_spec, _row_spec(), _row_spec(), _row_spec(),
                  _row_spec(), _row_spec(), _dinv_spec, _W_spec, _b_spec],
        out_specs=[_row_spec(), _row_spec()],
        out_shape=[jax.ShapeDtypeStruct((_N, _D), jnp.float32)] * 2,
    )(scal, m1, t0, t1, l0p, l1p, dinv, W, b)


def _stageE(l0, l1q, u0, u1, dinv, W, b, W0, b0, W1, b1, W2, b2, scal):
    def body(scal_ref, l0_ref, l1q_ref, u0_ref, u1_ref, dinv_ref, W_ref, b_ref,
             W0_ref, b0_ref, W1_ref, b1_ref, W2_ref, b2_ref, y_ref):
        l0 = l0_ref[...]
        sl0n = (u0_ref[...] + u1_ref[...]) * dinv_ref[...]
        dl0 = _mm_relu(l0, W_ref[...], b_ref[...])
        l1 = (l1q_ref[...] + scal_ref[0] * l0 + scal_ref[1] * dl0
              + scal_ref[2] * sl0n)
        hout = 0.5 * (l0 + l1)
        y0 = _mm_relu(hout, W0_ref[...], b0_ref[...])
        y1 = _mm_relu(y0, W1_ref[...], b1_ref[...])
        y_ref[...] = jnp.dot(y1, W2_ref[...],
                             preferred_element_type=jnp.float32) + b2_ref[...]

    return pl.pallas_call(
        body,
        grid=(_GRID,),
        in_specs=[
            _scal_spec, _row_spec(), _row_spec(), _row_spec(), _row_spec(),
            _dinv_spec, _W_spec, _b_spec,
            pl.BlockSpec((_D, _D // 2), lambda i: (0, 0)),
            pl.BlockSpec((1, _D // 2), lambda i: (0, 0)),
            pl.BlockSpec((_D // 2, _D // 4), lambda i: (0, 0)),
            pl.BlockSpec((1, _D // 4), lambda i: (0, 0)),
            pl.BlockSpec((_D // 4, _C), lambda i: (0, 0)),
            pl.BlockSpec((1, _C), lambda i: (0, 0)),
        ],
        out_specs=pl.BlockSpec((_BN, _C), lambda i: (i, 0)),
        out_shape=jax.ShapeDtypeStruct((_N, _C), jnp.float32),
    )(scal, l0, l1q, u0, u1, dinv, W, b, W0, b0, W1, b1, W2, b2)


def kernel(h, edge_index, emb_table, alphas_first, alphas_middle, alphas_last,
           W_dense, b_dense, W_fc0, b_fc0, W_fc1, b_fc1, W_fc2, b_fc2):
    h = h.astype(jnp.int32)
    pad = _EPR * 128 - _E
    ar = jnp.arange(pad, dtype=jnp.int32)
    src1 = jnp.concatenate([edge_index[0].astype(jnp.int32), ar % _N])
    dst1 = jnp.concatenate([edge_index[1].astype(jnp.int32),
                            _N + (ar % (_APAD - _N))])
    src2 = src1.reshape(_EPR, 128)
    dst2 = dst1.reshape(_EPR, 128)
    ah = jnp.arange(_APAD - _N, dtype=jnp.int32)
    h1 = jnp.concatenate([h, ah % _N])

    Wf = jax.nn.softmax(alphas_first, axis=1)
    Wm = jax.nn.softmax(alphas_middle, axis=1)
    Wl = jax.nn.softmax(alphas_last, axis=1)

    xp = _emb_gather(h1, emb_table)
    x = xp[:_N]
    degp = _deg_hist(dst2)
    dinv = (1.0 / jnp.maximum(degp[0, :_N] + degp[1, :_N], 1.0))[:, None]
    b2d = b_dense[None, :]

    sx = _segsum(x, src2, dst2)
    scalA = jnp.stack([Wf[0, 1], Wf[0, 2], Wf[0, 3],
                       Wf[1, 1], Wf[1, 2], Wf[1, 3]])
    s1, s2p = _stageA(x, sx[0], sx[1], dinv, W_dense, b2d, scalA)

    ss1 = _segsum(s1, src2, dst2)
    scalB = jnp.stack([Wf[2, 1], Wf[2, 2], Wf[2, 3], Wm[0, 0], Wm[0, 1]])
    s2, m0 = _stageB(s1, ss1[0], ss1[1], dinv, s2p, W_dense, b2d, scalB)

    ss2 = _segsum(s2, src2, dst2)
    sm0 = _segsum(m0, src2, dst2)
    scalC = jnp.stack([Wm[1, 0], Wm[1, 1], Wl[0, 1], Wl[0, 2], Wl[0, 3],
                       Wl[2, 1], Wl[2, 2], Wl[2, 3]])
    m1, l0p, l1p = _stageC(s2, ss2[0], ss2[1], sm0[0], sm0[1], m0, dinv,
                           W_dense, b2d, scalC)

    sm1 = _segsum(m1, src2, dst2)
    scalD = jnp.stack([Wl[1, 1], Wl[1, 2], Wl[1, 3],
                       Wl[3, 1], Wl[3, 2], Wl[3, 3]])
    l0, l1q = _stageD(m1, sm1[0], sm1[1], l0p, l1p, dinv, W_dense, b2d, scalD)

    sl0 = _segsum(l0, src2, dst2)
    scalE = jnp.stack([Wl[4, 1], Wl[4, 2], Wl[4, 3]])
    y = _stageE(l0, l1q, sl0[0], sl0[1], dinv, W_dense, b2d,
                W_fc0, b_fc0[None, :], W_fc1, b_fc1[None, :],
                W_fc2, b_fc2[None, :], scalE)
    return y


# speedup vs baseline: 11.6724x; 1.0649x over previous
import functools

import jax
import jax.numpy as jnp
from jax import lax
from jax.experimental import pallas as pl
from jax.experimental.pallas import tpu as pltpu
from jax.experimental.pallas import tpu_sc as plsc

_N = 10000
_E = 320000
_D = 128
_C = 40
_NSC = 2
_NT = 16
_NW = _NSC * _NT
_EPR = 2560
_RPW = _EPR // _NW
_CH = 64
_NBUF = 4
_NSPLIT = 4
_CPW = _EPR * 128 // _CH // _NW
_APAD = 10240
_APT = _APAD // _NT


@functools.lru_cache(maxsize=None)
def _mesh():
    return plsc.VectorSubcoreMesh(
        core_axis_name="c", subcore_axis_name="s",
        num_cores=_NSC, num_subcores=_NT)


def _emb_gather(h1, emb):
    @functools.partial(
        pl.kernel,
        out_type=jax.ShapeDtypeStruct((_APAD, _D), jnp.float32),
        mesh=_mesh(),
        scratch_types=[
            pltpu.VMEM((128,), jnp.int32),
            pltpu.VMEM((128, _D), jnp.float32),
            pltpu.SemaphoreType.DMA,
        ],
    )
    def k(h1_hbm, emb_hbm, out_hbm, idxv, rows, sem):
        cid = lax.axis_index("c")
        sid = lax.axis_index("s")
        wid = sid * _NSC + cid

        def do(kk):
            pltpu.sync_copy(h1_hbm.at[pl.ds(kk * 128, 128)], idxv)
            pltpu.async_copy(emb_hbm.at[idxv], rows, sem).wait()
            pltpu.sync_copy(rows, out_hbm.at[pl.ds(kk * 128, 128)])

        do(wid)
        do(wid + _NW)

        @pl.when(wid < (_APAD // 128) - 2 * _NW)
        def _():
            do(wid + 2 * _NW)

    return k(h1, emb)


def _deg_hist(dst2):
    @functools.partial(
        pl.kernel,
        out_type=jax.ShapeDtypeStruct((_NSC, _APAD), jnp.float32),
        mesh=_mesh(),
        scratch_types=[
            pltpu.VMEM((_RPW, 128), jnp.int32),
            pltpu.VMEM((128,), jnp.float32),
            pltpu.VMEM((_APT,), jnp.float32),
            pltpu.VMEM_SHARED((_APAD,), jnp.float32),
        ],
    )
    def k(dst_hbm, out_hbm, dstv, onesv, zb, dacc):
        cid = lax.axis_index("c")
        sid = lax.axis_index("s")
        wid = sid * _NSC + cid
        ones16 = jnp.full((16,), 1.0, jnp.float32)
        z16 = jnp.zeros((16,), jnp.float32)

        def f1(i, _):
            onesv[pl.ds(i * 16, 16)] = ones16
            return 0

        lax.fori_loop(0, 128 // 16, f1, 0)

        def f0(i, _):
            zb[pl.ds(i * 16, 16)] = z16
            return 0

        lax.fori_loop(0, _APT // 16, f0, 0)
        pltpu.sync_copy(zb, dacc.at[pl.ds(sid * _APT, _APT)])
        plsc.subcore_barrier()

        pltpu.sync_copy(dst_hbm.at[pl.ds(wid * _RPW, _RPW)], dstv)

        def body(j, _):
            pltpu.sync_copy(onesv, dacc.at[dstv.at[j]], add=True)
            return 0

        lax.fori_loop(0, _RPW, body, 0)

        plsc.subcore_barrier()
        pltpu.sync_copy(dacc.at[pl.ds(sid * _APT, _APT)],
                        out_hbm.at[cid, pl.ds(sid * _APT, _APT)])

    return k(dst2)


def _segsum(z, src2, dst2):
    @functools.partial(
        pl.kernel,
        out_type=jax.ShapeDtypeStruct((_NSC, _N, _D), jnp.float32),
        mesh=_mesh(),
        scratch_types=[
            pltpu.VMEM((_CPW // _NSPLIT, _CH), jnp.int32),
            pltpu.VMEM((_CPW // _NSPLIT, _CH), jnp.int32),
            [pltpu.VMEM((_CH, _D), jnp.float32)] * _NBUF,
            [pltpu.SemaphoreType.DMA] * _NBUF,
            [pltpu.SemaphoreType.DMA] * _NBUF,
            pltpu.VMEM_SHARED((_APAD, _D), jnp.float32),
        ],
    )
    def k(z_hbm, src_hbm, dst_hbm, out_hbm, srcv, dstv, rows, semg, sems, acc):
        cid = lax.axis_index("c")
        sid = lax.axis_index("s")
        wid = sid * _NSC + cid
        z16 = jnp.zeros((16,), jnp.float32)

        def zf(i, _):
            rows[0][i // (_D // 16), pl.ds((i % (_D // 16)) * 16, 16)] = z16
            return 0

        lax.fori_loop(0, _CH * (_D // 16), zf, 0)
        for t in range(_APT // _CH):
            pltpu.sync_copy(rows[0], acc.at[pl.ds(sid * _APT + t * _CH, _CH)])
        plsc.subcore_barrier()

        half_rows = _CPW // _NSPLIT
        nq = half_rows // _NBUF
        for half in range(_NSPLIT):
            rbase = wid * _CPW + half * half_rows
            pltpu.sync_copy(src_hbm.at[pl.ds(rbase, half_rows)], srcv)
            pltpu.sync_copy(dst_hbm.at[pl.ds(rbase, half_rows)], dstv)
            for b in range(_NBUF):
                pltpu.async_copy(z_hbm.at[srcv.at[b]], rows[b], semg[b])

            def body(g, _):
                j0 = g * _NBUF
                for b in range(_NBUF):
                    pltpu.make_async_copy(z_hbm.at[srcv.at[j0 + b]], rows[b],
                                          semg[b]).wait()
                    pltpu.async_copy(rows[b], acc.at[dstv.at[j0 + b]],
                                     sems[b], add=True)

                    @pl.when(g < nq - 1)
                    def _():
                        pltpu.make_async_copy(rows[b], acc.at[dstv.at[0]],
                                              sems[b]).wait()
                        pltpu.async_copy(z_hbm.at[srcv.at[j0 + b + _NBUF]],
                                         rows[b], semg[b])
                return 0

            lax.fori_loop(0, nq, body, 0)
            for b in range(_NBUF):
                pltpu.make_async_copy(rows[b], acc.at[dstv.at[0]],
                                      sems[b]).wait()

        plsc.subcore_barrier()
        @pl.when(sid < _NT - 1)
        def _():
            pltpu.sync_copy(acc.at[pl.ds(sid * _APT, _APT)],
                            out_hbm.at[cid, pl.ds(sid * _APT, _APT)])

        @pl.when(sid == _NT - 1)
        def _():
            pltpu.sync_copy(acc.at[pl.ds((_NT - 1) * _APT, _N - (_NT - 1) * _APT)],
                            out_hbm.at[cid, pl.ds((_NT - 1) * _APT,
                                                  _N - (_NT - 1) * _APT)])

    return k(z, src2, dst2)


_BN = 1000
_GRID = _N // _BN


def _row_spec():
    return pl.BlockSpec((_BN, _D), lambda i: (i, 0))


_W_spec = pl.BlockSpec((_D, _D), lambda i: (0, 0))
_b_spec = pl.BlockSpec((1, _D), lambda i: (0, 0))
_dinv_spec = pl.BlockSpec((_BN, 1), lambda i: (i, 0))
_scal_spec = pl.BlockSpec(memory_space=pltpu.MemorySpace.SMEM)


def _mm_relu(z, W, b):
    return jnp.maximum(jnp.dot(z, W, preferred_element_type=jnp.float32) + b, 0.0)


def _stageA(x, p0, p1, dinv, W, b, scal):
    def body(scal_ref, x_ref, p0_ref, p1_ref, dinv_ref, W_ref, b_ref,
             s1_ref, s2p_ref):
        x = x_ref[...]
        dx = _mm_relu(x, W_ref[...], b_ref[...])
        sxn = (p0_ref[...] + p1_ref[...]) * dinv_ref[...]
        s1_ref[...] = scal_ref[0] * x + scal_ref[1] * dx + scal_ref[2] * sxn
        s2p_ref[...] = scal_ref[3] * x + scal_ref[4] * dx + scal_ref[5] * sxn

    return pl.pallas_call(
        body,
        grid=(_GRID,),
        in_specs=[_scal_spec, _row_spec(), _row_spec(), _row_spec(),
                  _dinv_spec, _W_spec, _b_spec],
        out_specs=[_row_spec(), _row_spec()],
        out_shape=[jax.ShapeDtypeStruct((_N, _D), jnp.float32)] * 2,
    )(scal, x, p0, p1, dinv, W, b)


def _stageB(s1, p0, p1, dinv, s2p, W, b, scal):
    def body(scal_ref, s1_ref, p0_ref, p1_ref, dinv_ref, s2p_ref, W_ref, b_ref,
             s2_ref, m0_ref):
        s1 = s1_ref[...]
        ds1 = _mm_relu(s1, W_ref[...], b_ref[...])
        ss1n = (p0_ref[...] + p1_ref[...]) * dinv_ref[...]
        s2_ref[...] = (s2p_ref[...] + scal_ref[0] * s1 + scal_ref[1] * ds1
                       + scal_ref[2] * ss1n)
        m0_ref[...] = scal_ref[3] * s1 + scal_ref[4] * ss1n

    return pl.pallas_call(
        body,
        grid=(_GRID,),
        in_specs=[_scal_spec, _row_spec(), _row_spec(), _row_spec(),
                  _dinv_spec, _row_spec(), _W_spec, _b_spec],
        out_specs=[_row_spec(), _row_spec()],
        out_shape=[jax.ShapeDtypeStruct((_N, _D), jnp.float32)] * 2,
    )(scal, s1, p0, p1, dinv, s2p, W, b)


def _stageC(s2, q0, q1, r0, r1, m0, dinv, W, b, scal):
    def body(scal_ref, s2_ref, q0_ref, q1_ref, r0_ref, r1_ref, m0_ref,
             dinv_ref, W_ref, b_ref, m1_ref, l0p_ref, l1p_ref):
        s2 = s2_ref[...]
        m0 = m0_ref[...]
        dinv = dinv_ref[...]
        ss2n = (q0_ref[...] + q1_ref[...]) * dinv
        sm0n = (r0_ref[...] + r1_ref[...]) * dinv
        dm0 = _mm_relu(m0, W_ref[...], b_ref[...])
        m1_ref[...] = scal_ref[0] * s2 + scal_ref[1] * ss2n
        l0p_ref[...] = scal_ref[2] * m0 + scal_ref[3] * dm0 + scal_ref[4] * sm0n
        l1p_ref[...] = scal_ref[5] * m0 + scal_ref[6] * dm0 + scal_ref[7] * sm0n

    return pl.pallas_call(
        body,
        grid=(_GRID,),
        in_specs=[_scal_spec, _row_spec(), _row_spec(), _row_spec(),
                  _row_spec(), _row_spec(), _row_spec(), _dinv_spec,
                  _W_spec, _b_spec],
        out_specs=[_row_spec(), _row_spec(), _row_spec()],
        out_shape=[jax.ShapeDtypeStruct((_N, _D), jnp.float32)] * 3,
    )(scal, s2, q0, q1, r0, r1, m0, dinv, W, b)


def _stageD(m1, t0, t1, l0p, l1p, dinv, W, b, scal):
    def body(scal_ref, m1_ref, t0_ref, t1_ref, l0p_ref, l1p_ref, dinv_ref,
             W_ref, b_ref, l0_ref, l1q_ref):
        m1 = m1_ref[...]
        sm1n = (t0_ref[...] + t1_ref[...]) * dinv_ref[...]
        dm1 = _mm_relu(m1, W_ref[...], b_ref[...])
        l0_ref[...] = (l0p_ref[...] + scal_ref[0] * m1 + scal_ref[1] * dm1
                       + scal_ref[2] * sm1n)
        l1q_ref[...] = (l1p_ref[...] + scal_ref[3] * m1 + scal_ref[4] * dm1
                        + scal_ref[5] * sm1n)

    return pl.pallas_call(
        body,
        grid=(_GRID,),
        in_specs=[_scal_spec, _row_spec(), _row_spec(), _row_spec(),
                  _row_spec(), _row_spec(), _dinv_spec, _W_spec, _b_spec],
        out_specs=[_row_spec(), _row_spec()],
        out_shape=[jax.ShapeDtypeStruct((_N, _D), jnp.float32)] * 2,
    )(scal, m1, t0, t1, l0p, l1p, dinv, W, b)


def _stageE(l0, l1q, u0, u1, dinv, W, b, W0, b0, W1, b1, W2, b2, scal):
    def body(scal_ref, l0_ref, l1q_ref, u0_ref, u1_ref, dinv_ref, W_ref, b_ref,
             W0_ref, b0_ref, W1_ref, b1_ref, W2_ref, b2_ref, y_ref):
        l0 = l0_ref[...]
        sl0n = (u0_ref[...] + u1_ref[...]) * dinv_ref[...]
        dl0 = _mm_relu(l0, W_ref[...], b_ref[...])
        l1 = (l1q_ref[...] + scal_ref[0] * l0 + scal_ref[1] * dl0
              + scal_ref[2] * sl0n)
        hout = 0.5 * (l0 + l1)
        y0 = _mm_relu(hout, W0_ref[...], b0_ref[...])
        y1 = _mm_relu(y0, W1_ref[...], b1_ref[...])
        y_ref[...] = jnp.dot(y1, W2_ref[...],
                             preferred_element_type=jnp.float32) + b2_ref[...]

    return pl.pallas_call(
        body,
        grid=(_GRID,),
        in_specs=[
            _scal_spec, _row_spec(), _row_spec(), _row_spec(), _row_spec(),
            _dinv_spec, _W_spec, _b_spec,
            pl.BlockSpec((_D, _D // 2), lambda i: (0, 0)),
            pl.BlockSpec((1, _D // 2), lambda i: (0, 0)),
            pl.BlockSpec((_D // 2, _D // 4), lambda i: (0, 0)),
            pl.BlockSpec((1, _D // 4), lambda i: (0, 0)),
            pl.BlockSpec((_D // 4, _C), lambda i: (0, 0)),
            pl.BlockSpec((1, _C), lambda i: (0, 0)),
        ],
        out_specs=pl.BlockSpec((_BN, _C), lambda i: (i, 0)),
        out_shape=jax.ShapeDtypeStruct((_N, _C), jnp.float32),
    )(scal, l0, l1q, u0, u1, dinv, W, b, W0, b0, W1, b1, W2, b2)


def kernel(h, edge_index, emb_table, alphas_first, alphas_middle, alphas_last,
           W_dense, b_dense, W_fc0, b_fc0, W_fc1, b_fc1, W_fc2, b_fc2):
    h = h.astype(jnp.int32)
    pad = _EPR * 128 - _E
    ar = jnp.arange(pad, dtype=jnp.int32)
    src1 = jnp.concatenate([edge_index[0].astype(jnp.int32), ar % _N])
    dst1 = jnp.concatenate([edge_index[1].astype(jnp.int32),
                            _N + (ar % (_APAD - _N))])
    src64 = src1.reshape(-1, _CH)
    dst2 = dst1.reshape(_EPR, 128)
    dst64 = dst1.reshape(-1, _CH)
    ah = jnp.arange(_APAD - _N, dtype=jnp.int32)
    h1 = jnp.concatenate([h, ah % _N])

    Wf = jax.nn.softmax(alphas_first, axis=1)
    Wm = jax.nn.softmax(alphas_middle, axis=1)
    Wl = jax.nn.softmax(alphas_last, axis=1)

    xp = _emb_gather(h1, emb_table)
    x = xp[:_N]
    degp = _deg_hist(dst2)
    dinv = (1.0 / jnp.maximum(degp[0, :_N] + degp[1, :_N], 1.0))[:, None]
    b2d = b_dense[None, :]

    sx = _segsum(x, src64, dst64)
    scalA = jnp.stack([Wf[0, 1], Wf[0, 2], Wf[0, 3],
                       Wf[1, 1], Wf[1, 2], Wf[1, 3]])
    s1, s2p = _stageA(x, sx[0], sx[1], dinv, W_dense, b2d, scalA)

    ss1 = _segsum(s1, src64, dst64)
    scalB = jnp.stack([Wf[2, 1], Wf[2, 2], Wf[2, 3], Wm[0, 0], Wm[0, 1]])
    s2, m0 = _stageB(s1, ss1[0], ss1[1], dinv, s2p, W_dense, b2d, scalB)

    ss2 = _segsum(s2, src64, dst64)
    sm0 = _segsum(m0, src64, dst64)
    scalC = jnp.stack([Wm[1, 0], Wm[1, 1], Wl[0, 1], Wl[0, 2], Wl[0, 3],
                       Wl[2, 1], Wl[2, 2], Wl[2, 3]])
    m1, l0p, l1p = _stageC(s2, ss2[0], ss2[1], sm0[0], sm0[1], m0, dinv,
                           W_dense, b2d, scalC)

    sm1 = _segsum(m1, src64, dst64)
    scalD = jnp.stack([Wl[1, 1], Wl[1, 2], Wl[1, 3],
                       Wl[3, 1], Wl[3, 2], Wl[3, 3]])
    l0, l1q = _stageD(m1, sm1[0], sm1[1], l0p, l1p, dinv, W_dense, b2d, scalD)

    sl0 = _segsum(l0, src64, dst64)
    scalE = jnp.stack([Wl[4, 1], Wl[4, 2], Wl[4, 3]])
    y = _stageE(l0, l1q, sl0[0], sl0[1], dinv, W_dense, b2d,
                W_fc0, b_fc0[None, :], W_fc1, b_fc1[None, :],
                W_fc2, b_fc2[None, :], scalE)
    return y
